# baseline (speedup 1.0000x reference)
import functools

import jax
import jax.numpy as jnp
from jax import lax
from jax.experimental import pallas as pl
from jax.experimental.pallas import tpu as pltpu
from jax.experimental.pallas import tpu_sc as plsc

_B = 16384
_E = 32
_PACK = 128 // _E
_NC = 2
_NS = 16
_NW = _NC * _NS
_BPW = _B // _NW

_TC_BLK = 2048


def _sc_gather_pair(user_packed, item_packed, uhi, ihi):
    mesh = plsc.VectorSubcoreMesh(core_axis_name="c", subcore_axis_name="s")

    @functools.partial(
        pl.kernel,
        mesh=mesh,
        out_type=(
            jax.ShapeDtypeStruct((_B, 128), jnp.float32),
            jax.ShapeDtypeStruct((_B, 128), jnp.float32),
        ),
        scratch_types=[
            pltpu.VMEM((_BPW,), jnp.int32),
            pltpu.VMEM((_BPW, 128), jnp.float32),
            pltpu.SemaphoreType.DMA,
        ],
    )
    def gather_kernel(ut_hbm, it_hbm, ui_hbm, ii_hbm, uo_hbm, io_hbm,
                      idx_v, rows_v, sem):
        wid = lax.axis_index("s") * _NC + lax.axis_index("c")
        base = wid * _BPW
        pltpu.sync_copy(ui_hbm.at[pl.ds(base, _BPW)], idx_v)
        pltpu.async_copy(ut_hbm.at[idx_v], rows_v, sem).wait()
        pltpu.sync_copy(rows_v, uo_hbm.at[pl.ds(base, _BPW)])
        pltpu.sync_copy(ii_hbm.at[pl.ds(base, _BPW)], idx_v)
        pltpu.async_copy(it_hbm.at[idx_v], rows_v, sem).wait()
        pltpu.sync_copy(rows_v, io_hbm.at[pl.ds(base, _BPW)])

    return gather_kernel(user_packed, item_packed, uhi, ihi)


def _select_sub(g, sub):
    out = jnp.where(sub == 0, g[:, 0 * _E:1 * _E], 0.0)
    out += jnp.where(sub == 1, g[:, 1 * _E:2 * _E], 0.0)
    out += jnp.where(sub == 2, g[:, 2 * _E:3 * _E], 0.0)
    out += jnp.where(sub == 3, g[:, 3 * _E:4 * _E], 0.0)
    return out


def _tower(x, W0, b0, W1, b1, W2, b2):
    h = jnp.dot(x, W0, preferred_element_type=jnp.float32,
                precision=lax.Precision.HIGHEST) + b0
    h = jnp.maximum(h, 0.0)
    h = jnp.dot(h, W1, preferred_element_type=jnp.float32,
                precision=lax.Precision.HIGHEST) + b1
    h = jnp.maximum(h, 0.0)
    return jnp.dot(h, W2, preferred_element_type=jnp.float32,
                   precision=lax.Precision.HIGHEST) + b2


def _tc_body(ug_ref, ig_ref, us_ref, is_ref,
             uW0r, ub0r, uW1r, ub1r, uW2r, ub2r,
             iW0r, ib0r, iW1r, ib1r, iW2r, ib2r, o_ref):
    u_emb = _select_sub(ug_ref[...], us_ref[...])
    i_emb = _select_sub(ig_ref[...], is_ref[...])
    u = _tower(u_emb, uW0r[...], ub0r[...], uW1r[...], ub1r[...],
               uW2r[...], ub2r[...])
    v = _tower(i_emb, iW0r[...], ib0r[...], iW1r[...], ib1r[...],
               iW2r[...], ib2r[...])
    dot = jnp.sum(u * v, axis=1)
    nu = jnp.sqrt(jnp.sum(u * u, axis=1))
    nv = jnp.sqrt(jnp.sum(v * v, axis=1))
    o_ref[...] = dot / (jnp.maximum(nu, 1e-12) * jnp.maximum(nv, 1e-12))


def _tc_towers(u_g, i_g, u_sub, i_sub,
               uW0, ub0, uW1, ub1, uW2, ub2,
               iW0, ib0, iW1, ib1, iW2, ib2):
    def full(a):
        return pl.BlockSpec(a.shape, lambda i: (0,) * a.ndim)

    weights = [uW0, ub0, uW1, ub1, uW2, ub2, iW0, ib0, iW1, ib1, iW2, ib2]
    weights = [w.reshape(1, -1) if w.ndim == 1 else w for w in weights]
    return pl.pallas_call(
        _tc_body,
        grid=(_B // _TC_BLK,),
        in_specs=[
            pl.BlockSpec((_TC_BLK, 128), lambda i: (i, 0)),
            pl.BlockSpec((_TC_BLK, 128), lambda i: (i, 0)),
            pl.BlockSpec((_TC_BLK, 1), lambda i: (i, 0)),
            pl.BlockSpec((_TC_BLK, 1), lambda i: (i, 0)),
        ] + [full(w) for w in weights],
        out_specs=pl.BlockSpec((_TC_BLK,), lambda i: (i,)),
        out_shape=jax.ShapeDtypeStruct((_B,), jnp.float32),
    )(u_g, i_g, u_sub, i_sub, *weights)


@jax.jit
def kernel(user_idx, item_idx, user_table, item_table,
           uW0, ub0, uW1, ub1, uW2, ub2,
           iW0, ib0, iW1, ib1, iW2, ib2):
    ui = user_idx.astype(jnp.int32)
    ii = item_idx.astype(jnp.int32)
    up = user_table.reshape(-1, 128)
    ip = item_table.reshape(-1, 128)
    u_g, i_g = _sc_gather_pair(up, ip, ui >> 2, ii >> 2)
    u_sub = (ui & 3).reshape(_B, 1)
    i_sub = (ii & 3).reshape(_B, 1)
    return _tc_towers(u_g, i_g, u_sub, i_sub,
                      uW0, ub0, uW1, ub1, uW2, ub2,
                      iW0, ib0, iW1, ib1, iW2, ib2)


# speedup vs baseline: 1.0156x; 1.0156x over previous
import functools

import jax
import jax.numpy as jnp
from jax import lax
from jax.experimental import pallas as pl
from jax.experimental.pallas import tpu as pltpu
from jax.experimental.pallas import tpu_sc as plsc

_B = 16384
_E = 32
_PACK = 128 // _E
_NC = 2
_NS = 16
_NW = _NC * _NS
_BPW = _B // _NW

_TC_BLK = 2048


def _sc_gather_pair(user_table, item_table, ui, ii):
    mesh = plsc.VectorSubcoreMesh(core_axis_name="c", subcore_axis_name="s")

    @functools.partial(
        pl.kernel,
        mesh=mesh,
        out_type=(
            jax.ShapeDtypeStruct((_B, _E), jnp.float32),
            jax.ShapeDtypeStruct((_B, _E), jnp.float32),
        ),
        scratch_types=[
            pltpu.VMEM((_BPW,), jnp.int32),
            pltpu.VMEM((_BPW, _E), jnp.float32),
            pltpu.VMEM((_BPW,), jnp.int32),
            pltpu.VMEM((_BPW, _E), jnp.float32),
            pltpu.SemaphoreType.DMA,
            pltpu.SemaphoreType.DMA,
        ],
        compiler_params=pltpu.CompilerParams(use_tc_tiling_on_sc=False),
    )
    def gather_kernel(ut_hbm, it_hbm, ui_hbm, ii_hbm, uo_hbm, io_hbm,
                      uidx_v, urows_v, iidx_v, irows_v, usem, isem):
        wid = lax.axis_index("s") * _NC + lax.axis_index("c")
        base = wid * _BPW
        pltpu.sync_copy(ui_hbm.at[pl.ds(base, _BPW)], uidx_v)
        ucopy = pltpu.async_copy(ut_hbm.at[uidx_v], urows_v, usem)
        pltpu.sync_copy(ii_hbm.at[pl.ds(base, _BPW)], iidx_v)
        icopy = pltpu.async_copy(it_hbm.at[iidx_v], irows_v, isem)
        ucopy.wait()
        pltpu.sync_copy(urows_v, uo_hbm.at[pl.ds(base, _BPW)])
        icopy.wait()
        pltpu.sync_copy(irows_v, io_hbm.at[pl.ds(base, _BPW)])

    return gather_kernel(user_table, item_table, ui, ii)


def _tower(x, W0, b0, W1, b1, W2, b2):
    h = jnp.dot(x, W0, preferred_element_type=jnp.float32,
                precision=lax.Precision.HIGHEST) + b0
    h = jnp.maximum(h, 0.0)
    h = jnp.dot(h, W1, preferred_element_type=jnp.float32,
                precision=lax.Precision.HIGHEST) + b1
    h = jnp.maximum(h, 0.0)
    return jnp.dot(h, W2, preferred_element_type=jnp.float32,
                   precision=lax.Precision.HIGHEST) + b2


def _tc_body(ug_ref, ig_ref,
             uW0r, ub0r, uW1r, ub1r, uW2r, ub2r,
             iW0r, ib0r, iW1r, ib1r, iW2r, ib2r, o_ref):
    u = _tower(ug_ref[...], uW0r[...], ub0r[...], uW1r[...], ub1r[...],
               uW2r[...], ub2r[...])
    v = _tower(ig_ref[...], iW0r[...], ib0r[...], iW1r[...], ib1r[...],
               iW2r[...], ib2r[...])
    dot = jnp.sum(u * v, axis=1)
    nu = jnp.sqrt(jnp.sum(u * u, axis=1))
    nv = jnp.sqrt(jnp.sum(v * v, axis=1))
    o_ref[...] = dot / (jnp.maximum(nu, 1e-12) * jnp.maximum(nv, 1e-12))


def _tc_towers(u_g, i_g,
               uW0, ub0, uW1, ub1, uW2, ub2,
               iW0, ib0, iW1, ib1, iW2, ib2):
    def full(a):
        return pl.BlockSpec(a.shape, lambda i: (0,) * a.ndim)

    weights = [uW0, ub0, uW1, ub1, uW2, ub2, iW0, ib0, iW1, ib1, iW2, ib2]
    weights = [w.reshape(1, -1) if w.ndim == 1 else w for w in weights]
    return pl.pallas_call(
        _tc_body,
        grid=(_B // _TC_BLK,),
        in_specs=[
            pl.BlockSpec((_TC_BLK, _E), lambda i: (i, 0)),
            pl.BlockSpec((_TC_BLK, _E), lambda i: (i, 0)),
        ] + [full(w) for w in weights],
        out_specs=pl.BlockSpec((_TC_BLK,), lambda i: (i,)),
        out_shape=jax.ShapeDtypeStruct((_B,), jnp.float32),
    )(u_g, i_g, *weights)


@jax.jit
def kernel(user_idx, item_idx, user_table, item_table,
           uW0, ub0, uW1, ub1, uW2, ub2,
           iW0, ib0, iW1, ib1, iW2, ib2):
    ui = user_idx.astype(jnp.int32)
    ii = item_idx.astype(jnp.int32)
    u_g, i_g = _sc_gather_pair(user_table, item_table, ui, ii)
    return _tc_towers(u_g, i_g,
                      uW0, ub0, uW1, ub1, uW2, ub2,
                      iW0, ib0, iW1, ib1, iW2, ib2)


# speedup vs baseline: 1.0978x; 1.0809x over previous
import functools

import jax
import jax.numpy as jnp
from jax import lax
from jax.experimental import pallas as pl
from jax.experimental.pallas import tpu as pltpu
from jax.experimental.pallas import tpu_sc as plsc

_B = 16384
_E = 32
_NC = 2
_NS = 16
_NW = _NC * _NS
_BPW = _B // _NW
_NCH = 8
_CH = _BPW // _NCH

_TC_BLK = 2048


def _sc_gather_pair(user_packed, item_packed, uhi, ihi):
    mesh = plsc.VectorSubcoreMesh(core_axis_name="c", subcore_axis_name="s")

    @functools.partial(
        pl.kernel,
        mesh=mesh,
        out_type=(
            jax.ShapeDtypeStruct((_B, 128), jnp.float32),
            jax.ShapeDtypeStruct((_B, 128), jnp.float32),
        ),
        scratch_types=[
            pltpu.VMEM((_BPW,), jnp.int32),
            pltpu.VMEM((_BPW, 128), jnp.float32),
            pltpu.SemaphoreType.DMA,
        ],
    )
    def gather_kernel(ut_hbm, it_hbm, ui_hbm, ii_hbm, uo_hbm, io_hbm,
                      idx_v, rows_v, sem):
        wid = lax.axis_index("s") * _NC + lax.axis_index("c")
        base = wid * _BPW

        def one_table(tab_hbm, i_hbm, o_hbm):
            pltpu.sync_copy(i_hbm.at[pl.ds(base, _BPW)], idx_v)
            for ch in range(_NCH):
                pltpu.async_copy(
                    tab_hbm.at[idx_v.at[pl.ds(ch * _CH, _CH)]],
                    rows_v.at[pl.ds(ch * _CH, _CH), :], sem)
            pltpu.make_async_copy(tab_hbm.at[pl.ds(0, _BPW)], rows_v,
                                  sem).wait()
            pltpu.sync_copy(rows_v, o_hbm.at[pl.ds(base, _BPW)])

        one_table(ut_hbm, ui_hbm, uo_hbm)
        one_table(it_hbm, ii_hbm, io_hbm)

    return gather_kernel(user_packed, item_packed, uhi, ihi)


def _select_sub(g, sub):
    out = jnp.where(sub == 0, g[:, 0 * _E:1 * _E], 0.0)
    out += jnp.where(sub == 1, g[:, 1 * _E:2 * _E], 0.0)
    out += jnp.where(sub == 2, g[:, 2 * _E:3 * _E], 0.0)
    out += jnp.where(sub == 3, g[:, 3 * _E:4 * _E], 0.0)
    return out


def _tower(x, W0, b0, W1, b1, W2, b2):
    def mm(v, W):
        return jnp.dot(v.astype(jnp.bfloat16), W.astype(jnp.bfloat16),
                       preferred_element_type=jnp.float32)
    h = jnp.maximum(mm(x, W0) + b0, 0.0)
    h = jnp.maximum(mm(h, W1) + b1, 0.0)
    return mm(h, W2) + b2


def _tc_body(ug_ref, ig_ref, us_ref, is_ref,
             uW0r, ub0r, uW1r, ub1r, uW2r, ub2r,
             iW0r, ib0r, iW1r, ib1r, iW2r, ib2r, o_ref):
    u_emb = _select_sub(ug_ref[...], us_ref[...])
    i_emb = _select_sub(ig_ref[...], is_ref[...])
    u = _tower(u_emb, uW0r[...], ub0r[...], uW1r[...], ub1r[...],
               uW2r[...], ub2r[...])
    v = _tower(i_emb, iW0r[...], ib0r[...], iW1r[...], ib1r[...],
               iW2r[...], ib2r[...])
    dot = jnp.sum(u * v, axis=1)
    nu = jnp.sqrt(jnp.sum(u * u, axis=1))
    nv = jnp.sqrt(jnp.sum(v * v, axis=1))
    o_ref[...] = dot / (jnp.maximum(nu, 1e-12) * jnp.maximum(nv, 1e-12))


def _tc_towers(u_g, i_g, u_sub, i_sub,
               uW0, ub0, uW1, ub1, uW2, ub2,
               iW0, ib0, iW1, ib1, iW2, ib2):
    def full(a):
        return pl.BlockSpec(a.shape, lambda i: (0,) * a.ndim)

    weights = [uW0, ub0, uW1, ub1, uW2, ub2, iW0, ib0, iW1, ib1, iW2, ib2]
    weights = [w.reshape(1, -1) if w.ndim == 1 else w for w in weights]
    return pl.pallas_call(
        _tc_body,
        grid=(_B // _TC_BLK,),
        in_specs=[
            pl.BlockSpec((_TC_BLK, 128), lambda i: (i, 0)),
            pl.BlockSpec((_TC_BLK, 128), lambda i: (i, 0)),
            pl.BlockSpec((_TC_BLK, 1), lambda i: (i, 0)),
            pl.BlockSpec((_TC_BLK, 1), lambda i: (i, 0)),
        ] + [full(w) for w in weights],
        out_specs=pl.BlockSpec((_TC_BLK,), lambda i: (i,)),
        out_shape=jax.ShapeDtypeStruct((_B,), jnp.float32),
    )(u_g, i_g, u_sub, i_sub, *weights)


@jax.jit
def kernel(user_idx, item_idx, user_table, item_table,
           uW0, ub0, uW1, ub1, uW2, ub2,
           iW0, ib0, iW1, ib1, iW2, ib2):
    ui = user_idx.astype(jnp.int32)
    ii = item_idx.astype(jnp.int32)
    up = user_table.reshape(-1, 128)
    ip = item_table.reshape(-1, 128)
    u_g, i_g = _sc_gather_pair(up, ip, ui >> 2, ii >> 2)
    u_sub = (ui & 3).reshape(_B, 1)
    i_sub = (ii & 3).reshape(_B, 1)
    return _tc_towers(u_g, i_g, u_sub, i_sub,
                      uW0, ub0, uW1, ub1, uW2, ub2,
                      iW0, ib0, iW1, ib1, iW2, ib2)
